# baseline (speedup 1.0000x reference)
import functools

import jax
import jax.numpy as jnp
from jax import lax
from jax.experimental import pallas as pl
from jax.experimental.pallas import tpu as pltpu
from jax.experimental.pallas import tpu_sc as plsc

NUM_EMB = 1000000
D = 64
B = 4096
L = 50

NC = 2
NS = 16
NW = NC * NS
BPW = B // NW
C = 8
NCHUNK = BPW // C
NG = D // 16

RB = 4096
QB = RB // 2
NBLK_T = (NUM_EMB + RB - 1) // RB
NQ = NBLK_T * QB


def _pack_kernel(wta_ref, wtb_ref, out_ref):
    out_ref[...] = jnp.concatenate(
        [wta_ref[...].T, wtb_ref[...].T], axis=1
    )


def _pack(wt):
    return pl.pallas_call(
        _pack_kernel,
        grid=(NBLK_T,),
        in_specs=[
            pl.BlockSpec((D, QB), lambda k: (0, 2 * k)),
            pl.BlockSpec(
                (D, QB),
                lambda k: (0, jnp.minimum(2 * k + 1, 2 * NBLK_T - 2)),
            ),
        ],
        out_specs=pl.BlockSpec((QB, 2 * D), lambda k: (k, 0)),
        out_shape=jax.ShapeDtypeStruct((NQ, 2 * D), jnp.float32),
    )(wt, wt)


def _emb_bag_kernel(qs_hbm, kb_hbm, w2_hbm, out_hbm,
                    idx_v, kb_v, rows_v, out_v, sem):
    wid = lax.axis_index("s") * NC + lax.axis_index("c")
    base = wid * BPW

    @pl.loop(0, NCHUNK)
    def _chunk(t):
        bag = base + t * C
        pltpu.sync_copy(qs_hbm.at[pl.ds(bag, C)], idx_v)
        pltpu.sync_copy(kb_hbm.at[pl.ds(bag, 16)], kb_v)
        copies = [
            pltpu.async_copy(w2_hbm.at[idx_v.at[c]], rows_v.at[c], sem)
            for c in range(C)
        ]
        kbs = kb_v[...]
        for cp in copies:
            cp.wait()
        for c in range(C):
            kbc = kbs[c]

            def body_even(j, accs):
                return tuple(
                    accs[g] + rows_v[c, j, pl.ds(g * 16, 16)]
                    for g in range(NG)
                )

            def body_odd(j, accs):
                return tuple(
                    accs[g] + rows_v[c, j, pl.ds(D + g * 16, 16)]
                    for g in range(NG)
                )

            z = tuple(jnp.zeros((16,), jnp.float32) for _ in range(NG))
            accs = lax.fori_loop(0, kbc, body_even, z)
            accs = lax.fori_loop(kbc, L, body_odd, accs)
            for g in range(NG):
                out_v[c, pl.ds(g * 16, 16)] = accs[g] * (1.0 / L)
        pltpu.sync_copy(out_v, out_hbm.at[pl.ds(bag, C)])


@jax.jit
def _emb_bag(x, weight):
    h = lax.shift_right_logical(lax.bitwise_and(x, RB - 1), 11)
    perm = jnp.argsort(h, axis=1, stable=True)
    xs = jnp.take_along_axis(x, perm, axis=1)
    qs = (lax.shift_right_logical(xs, 12) * QB
          + lax.bitwise_and(xs, QB - 1))
    kb = jnp.pad((L - h.sum(axis=1)).astype(jnp.int32), (0, 16))

    w2 = _pack(weight.T)

    mesh = plsc.VectorSubcoreMesh(core_axis_name="c", subcore_axis_name="s")
    f = pl.kernel(
        _emb_bag_kernel,
        out_type=jax.ShapeDtypeStruct((B, D), jnp.float32),
        mesh=mesh,
        scratch_types=[
            pltpu.VMEM((C, L), jnp.int32),
            pltpu.VMEM((16,), jnp.int32),
            pltpu.VMEM((C, L, 2 * D), jnp.float32),
            pltpu.VMEM((C, D), jnp.float32),
            pltpu.SemaphoreType.DMA,
        ],
        compiler_params=pltpu.CompilerParams(use_tc_tiling_on_sc=False),
    )
    return f(qs, kb, w2)


def kernel(x, weight):
    return _emb_bag(x.astype(jnp.int32), weight)


# speedup vs baseline: 1.2816x; 1.2816x over previous
import functools

import jax
import jax.numpy as jnp
from jax import lax
from jax.experimental import pallas as pl
from jax.experimental.pallas import tpu as pltpu
from jax.experimental.pallas import tpu_sc as plsc

NUM_EMB = 1000000
D = 64
B = 4096
L = 50

NC = 2
NS = 16
NW = NC * NS
BPW = B // NW
C = 8
NCHUNK = BPW // C
NG = D // 16

LOG_RB = 13
RB = 1 << LOG_RB
QB = RB // 2
LOG_QB = LOG_RB - 1
NBLK_T = (NUM_EMB + RB - 1) // RB
NQ = NBLK_T * QB


def _pack_kernel(wt_ref, out_ref):
    v = wt_ref[...]
    out_ref[...] = jnp.concatenate(
        [v[:, :QB].T, v[:, QB:].T], axis=1
    )


def _pack(wt):
    return pl.pallas_call(
        _pack_kernel,
        grid=(NBLK_T,),
        in_specs=[pl.BlockSpec((D, RB), lambda k: (0, k))],
        out_specs=pl.BlockSpec((QB, 2 * D), lambda k: (k, 0)),
        out_shape=jax.ShapeDtypeStruct((NQ, 2 * D), jnp.float32),
    )(wt)


def _fire_gathers(w2_hbm, idx_v, rows_v, sem, buf):
    return [
        pltpu.async_copy(
            w2_hbm.at[idx_v.at[buf, c]],
            rows_v.at[buf, pl.ds(c * L, L)],
            sem.at[buf],
        )
        for c in range(C)
    ]


def _emb_bag_kernel(qs_hbm, kb_hbm, w2_hbm, out_hbm,
                    idx_v, kb_v, rows_v, out_v, sem):
    wid = lax.axis_index("s") * NC + lax.axis_index("c")
    base = wid * BPW

    def _stage(t, buf):
        bag = base + t * C
        pltpu.sync_copy(qs_hbm.at[pl.ds(bag, C)], idx_v.at[buf])
        pltpu.sync_copy(kb_hbm.at[pl.ds(bag, 16)], kb_v.at[buf])
        _fire_gathers(w2_hbm, idx_v, rows_v, sem, buf)

    _stage(0, 0)

    @pl.loop(0, NCHUNK)
    def _chunk(t):
        buf = lax.rem(t, 2)
        bag = base + t * C

        @pl.when(t + 1 < NCHUNK)
        def _prefetch():
            _stage(t + 1, 1 - buf)

        pltpu.make_async_copy(
            w2_hbm.at[pl.ds(0, C * L)], rows_v.at[buf], sem.at[buf]
        ).wait()

        kbs = kb_v[buf]
        for c in range(C):
            kbc = kbs[c]

            def body_even(j, accs):
                return tuple(
                    accs[g] + rows_v[buf, c * L + j, pl.ds(g * 16, 16)]
                    for g in range(NG)
                )

            def body_odd(j, accs):
                return tuple(
                    accs[g] + rows_v[buf, c * L + j, pl.ds(D + g * 16, 16)]
                    for g in range(NG)
                )

            z = tuple(jnp.zeros((16,), jnp.float32) for _ in range(NG))
            accs = lax.fori_loop(0, kbc, body_even, z)
            accs = lax.fori_loop(kbc, L, body_odd, accs)
            for g in range(NG):
                out_v[c, pl.ds(g * 16, 16)] = accs[g] * (1.0 / L)
        pltpu.sync_copy(out_v, out_hbm.at[pl.ds(bag, C)])


@jax.jit
def _emb_bag(x, weight):
    h = lax.shift_right_logical(lax.bitwise_and(x, RB - 1), LOG_QB)
    perm = jnp.argsort(h, axis=1, stable=True)
    xs = jnp.take_along_axis(x, perm, axis=1)
    qs = (lax.shift_right_logical(xs, LOG_RB) * QB
          + lax.bitwise_and(xs, QB - 1))
    kb = jnp.pad((L - h.sum(axis=1)).astype(jnp.int32), (0, 16))

    w2 = _pack(weight.T)

    mesh = plsc.VectorSubcoreMesh(core_axis_name="c", subcore_axis_name="s")
    f = pl.kernel(
        _emb_bag_kernel,
        out_type=jax.ShapeDtypeStruct((B, D), jnp.float32),
        mesh=mesh,
        scratch_types=[
            pltpu.VMEM((2, C, L), jnp.int32),
            pltpu.VMEM((2, 16), jnp.int32),
            pltpu.VMEM((2, C * L, 2 * D), jnp.float32),
            pltpu.VMEM((C, D), jnp.float32),
            pltpu.SemaphoreType.DMA((2,)),
        ],
        compiler_params=pltpu.CompilerParams(use_tc_tiling_on_sc=False),
    )
    return f(qs, kb, w2)


def kernel(x, weight):
    return _emb_bag(x.astype(jnp.int32), weight)


# speedup vs baseline: 1.5552x; 1.2135x over previous
import functools

import jax
import jax.numpy as jnp
from jax import lax
from jax.experimental import pallas as pl
from jax.experimental.pallas import tpu as pltpu
from jax.experimental.pallas import tpu_sc as plsc

NUM_EMB = 1000000
D = 64
B = 4096
L = 50

NC = 2
NS = 16
NW = NC * NS
BPW = B // NW
C = 8
NCHUNK = BPW // C
NG = D // 16

LOG_RB = 14
RB = 1 << LOG_RB
QB = RB // 2
LOG_QB = LOG_RB - 1
NBLK_T = (NUM_EMB + RB - 1) // RB
NQ = NBLK_T * QB


def _pack_kernel(wt_ref, out_ref):
    v = wt_ref[...]
    out_ref[...] = jnp.concatenate(
        [v[:, :QB].T, v[:, QB:].T], axis=1
    )


def _pack(wt):
    return pl.pallas_call(
        _pack_kernel,
        grid=(NBLK_T,),
        in_specs=[pl.BlockSpec((D, RB), lambda k: (0, k))],
        out_specs=pl.BlockSpec((QB, 2 * D), lambda k: (k, 0)),
        out_shape=jax.ShapeDtypeStruct((NQ, 2 * D), jnp.float32),
    )(wt)


def _fire_gathers(w2_hbm, idx_v, rows_v, sem, buf):
    return [
        pltpu.async_copy(
            w2_hbm.at[idx_v.at[buf, c]],
            rows_v.at[buf, pl.ds(c * L, L)],
            sem.at[buf],
        )
        for c in range(C)
    ]


def _emb_bag_kernel(qs_hbm, ho_hbm, w2_hbm, out_hbm,
                    idx_v, ho_v, rows_v, out_v, sem):
    wid = lax.axis_index("s") * NC + lax.axis_index("c")
    base = wid * BPW

    def _stage(t, buf):
        bag = base + t * C
        pltpu.sync_copy(qs_hbm.at[pl.ds(bag, C)], idx_v.at[buf])
        pltpu.sync_copy(ho_hbm.at[pl.ds(bag, C)], ho_v.at[buf])
        _fire_gathers(w2_hbm, idx_v, rows_v, sem, buf)

    _stage(0, 0)

    @pl.loop(0, NCHUNK)
    def _chunk(t):
        buf = lax.rem(t, 2)
        bag = base + t * C

        @pl.when(t + 1 < NCHUNK)
        def _prefetch():
            _stage(t + 1, 1 - buf)

        pltpu.make_async_copy(
            w2_hbm.at[pl.ds(0, C * L)], rows_v.at[buf], sem.at[buf]
        ).wait()

        for c in range(C):
            def body(j, accs):
                off = ho_v[buf, c, pl.ds(j, 16)][0]
                return tuple(
                    accs[g]
                    + rows_v[buf, c * L + j, pl.ds(off + g * 16, 16)]
                    for g in range(NG)
                )

            z = tuple(jnp.zeros((16,), jnp.float32) for _ in range(NG))
            accs = lax.fori_loop(0, L, body, z)
            for g in range(NG):
                out_v[c, pl.ds(g * 16, 16)] = accs[g] * (1.0 / L)
        pltpu.sync_copy(out_v, out_hbm.at[pl.ds(bag, C)])


@jax.jit
def _emb_bag(x, weight):
    qs = (lax.shift_right_logical(x, LOG_RB) * QB
          + lax.bitwise_and(x, QB - 1))
    h = lax.shift_right_logical(lax.bitwise_and(x, RB - 1), LOG_QB)
    ho = jnp.pad(h * D, ((0, 0), (0, 72 - L)))

    w2 = _pack(weight.T)

    mesh = plsc.VectorSubcoreMesh(core_axis_name="c", subcore_axis_name="s")
    f = pl.kernel(
        _emb_bag_kernel,
        out_type=jax.ShapeDtypeStruct((B, D), jnp.float32),
        mesh=mesh,
        scratch_types=[
            pltpu.VMEM((2, C, L), jnp.int32),
            pltpu.VMEM((2, C, 72), jnp.int32),
            pltpu.VMEM((2, C * L, 2 * D), jnp.float32),
            pltpu.VMEM((C, D), jnp.float32),
            pltpu.SemaphoreType.DMA((2,)),
        ],
        compiler_params=pltpu.CompilerParams(use_tc_tiling_on_sc=False),
    )
    return f(qs, ho, w2)


def kernel(x, weight):
    return _emb_bag(x.astype(jnp.int32), weight)


# speedup vs baseline: 1.6256x; 1.0453x over previous
import functools

import jax
import jax.numpy as jnp
from jax import lax
from jax.experimental import pallas as pl
from jax.experimental.pallas import tpu as pltpu
from jax.experimental.pallas import tpu_sc as plsc

NUM_EMB = 1000000
D = 64
B = 4096
L = 50

NC = 2
NS = 16
NW = NC * NS
BPW = B // NW
C = 8
NCHUNK = BPW // C
NG = D // 16

LOG_RB = 15
RB = 1 << LOG_RB
QB = RB // 2
LOG_QB = LOG_RB - 1
NBLK_T = (NUM_EMB + RB - 1) // RB
NQ = NBLK_T * QB


def _pack_kernel(wt_ref, out_ref):
    v = wt_ref[...]
    out_ref[...] = jnp.concatenate(
        [v[:, :QB].T, v[:, QB:].T], axis=1
    )


def _pack(wt):
    return pl.pallas_call(
        _pack_kernel,
        grid=(NBLK_T,),
        in_specs=[pl.BlockSpec((D, RB), lambda k: (0, k))],
        out_specs=pl.BlockSpec((QB, 2 * D), lambda k: (k, 0)),
        out_shape=jax.ShapeDtypeStruct((NQ, 2 * D), jnp.float32),
    )(wt)


def _fire_gathers(w2_hbm, idx_v, rows_v, sem, buf):
    return [
        pltpu.async_copy(
            w2_hbm.at[idx_v.at[buf, c]],
            rows_v.at[buf, pl.ds(c * L, L)],
            sem.at[buf],
        )
        for c in range(C)
    ]


def _emb_bag_kernel(qs_hbm, ho_hbm, w2_hbm, out_hbm,
                    idx_v, ho_v, rows_v, out_v, sem):
    wid = lax.axis_index("s") * NC + lax.axis_index("c")
    base = wid * BPW

    def _stage(t, buf):
        bag = base + t * C
        pltpu.sync_copy(qs_hbm.at[pl.ds(bag, C)], idx_v.at[buf])
        pltpu.sync_copy(ho_hbm.at[pl.ds(bag, C)], ho_v.at[buf])
        _fire_gathers(w2_hbm, idx_v, rows_v, sem, buf)

    _stage(0, 0)

    @pl.loop(0, NCHUNK)
    def _chunk(t):
        buf = lax.rem(t, 2)
        bag = base + t * C

        @pl.when(t + 1 < NCHUNK)
        def _prefetch():
            _stage(t + 1, 1 - buf)

        pltpu.make_async_copy(
            w2_hbm.at[pl.ds(0, C * L)], rows_v.at[buf], sem.at[buf]
        ).wait()

        for c in range(C):
            def body(j, accs):
                off = ho_v[buf, c, pl.ds(j, 16)][0]
                return tuple(
                    accs[g]
                    + rows_v[buf, c * L + j, pl.ds(off + g * 16, 16)]
                    for g in range(NG)
                )

            z = tuple(jnp.zeros((16,), jnp.float32) for _ in range(NG))
            accs = lax.fori_loop(0, L, body, z)
            for g in range(NG):
                out_v[c, pl.ds(g * 16, 16)] = accs[g] * (1.0 / L)
        pltpu.sync_copy(out_v, out_hbm.at[pl.ds(bag, C)])


@jax.jit
def _emb_bag(x, weight):
    qs = (lax.shift_right_logical(x, LOG_RB) * QB
          + lax.bitwise_and(x, QB - 1))
    h = lax.shift_right_logical(lax.bitwise_and(x, RB - 1), LOG_QB)
    ho = jnp.pad(h * D, ((0, 0), (0, 72 - L)))

    w2 = _pack(weight.T)

    mesh = plsc.VectorSubcoreMesh(core_axis_name="c", subcore_axis_name="s")
    f = pl.kernel(
        _emb_bag_kernel,
        out_type=jax.ShapeDtypeStruct((B, D), jnp.float32),
        mesh=mesh,
        scratch_types=[
            pltpu.VMEM((2, C, L), jnp.int32),
            pltpu.VMEM((2, C, 72), jnp.int32),
            pltpu.VMEM((2, C * L, 2 * D), jnp.float32),
            pltpu.VMEM((C, D), jnp.float32),
            pltpu.SemaphoreType.DMA((2,)),
        ],
        compiler_params=pltpu.CompilerParams(use_tc_tiling_on_sc=False),
    )
    return f(qs, ho, w2)


def kernel(x, weight):
    return _emb_bag(x.astype(jnp.int32), weight)


# speedup vs baseline: 1.6257x; 1.0001x over previous
import jax
import jax.numpy as jnp
from jax import lax
from jax.experimental import pallas as pl
from jax.experimental.pallas import tpu as pltpu
from jax.experimental.pallas import tpu_sc as plsc

NUM_EMB = 1000000
D = 64
B = 4096
L = 50

NC = 2
NS = 16
NW = NC * NS
BPW = B // NW
C = 8
NCHUNK = BPW // C
NG = D // 16

LOG_RB = 15
RB = 1 << LOG_RB
QB = RB // 2
LOG_QB = LOG_RB - 1
NBLK_T = (NUM_EMB + RB - 1) // RB
NQ = NBLK_T * QB


def _pack_kernel(wt_ref, out_ref):
    v = wt_ref[...]
    out_ref[...] = jnp.concatenate(
        [v[:, :QB].T, v[:, QB:].T], axis=1
    )


def _pack(wt):
    return pl.pallas_call(
        _pack_kernel,
        grid=(NBLK_T,),
        in_specs=[pl.BlockSpec((D, RB), lambda k: (0, k))],
        out_specs=pl.BlockSpec((QB, 2 * D), lambda k: (k, 0)),
        out_shape=jax.ShapeDtypeStruct((NQ, 2 * D), jnp.float32),
    )(wt)


def _fire_gathers(w2_hbm, idx_v, rows_v, sem, buf):
    return [
        pltpu.async_copy(
            w2_hbm.at[idx_v.at[buf, c]],
            rows_v.at[buf, pl.ds(c * L, L)],
            sem.at[buf],
        )
        for c in range(C)
    ]


def _emb_bag_kernel(qs_hbm, ho_hbm, w2_hbm, out_hbm,
                    idx_v, ho_v, rows_v, out_v, sem):
    wid = lax.axis_index("s") * NC + lax.axis_index("c")
    base = wid * BPW

    def _stage(t, buf):
        bag = base + t * C
        pltpu.sync_copy(qs_hbm.at[pl.ds(bag, C)], idx_v.at[buf])
        pltpu.sync_copy(ho_hbm.at[pl.ds(bag, C)], ho_v.at[buf])
        _fire_gathers(w2_hbm, idx_v, rows_v, sem, buf)

    _stage(0, 0)

    @pl.loop(0, NCHUNK)
    def _chunk(t):
        buf = lax.rem(t, 2)
        bag = base + t * C

        @pl.when(t + 1 < NCHUNK)
        def _prefetch():
            _stage(t + 1, 1 - buf)

        pltpu.make_async_copy(
            w2_hbm.at[pl.ds(0, C * L)], rows_v.at[buf], sem.at[buf]
        ).wait()

        for c in range(C):
            def body(j, accs):
                off = ho_v[buf, c, pl.ds(j, 16)][0]
                return tuple(
                    accs[g]
                    + rows_v[buf, c * L + j, pl.ds(off + g * 16, 16)]
                    for g in range(NG)
                )

            z = tuple(jnp.zeros((16,), jnp.float32) for _ in range(NG))
            accs = lax.fori_loop(0, L, body, z)
            for g in range(NG):
                out_v[c, pl.ds(g * 16, 16)] = accs[g] * (1.0 / L)
        pltpu.sync_copy(out_v, out_hbm.at[pl.ds(bag, C)])


@jax.jit
def _emb_bag(x, weight):
    qs = (lax.shift_right_logical(x, LOG_RB) * QB
          + lax.bitwise_and(x, QB - 1))
    h = lax.shift_right_logical(lax.bitwise_and(x, RB - 1), LOG_QB)
    ho = jnp.pad(h * D, ((0, 0), (0, 72 - L)))

    w2 = _pack(weight.T)

    mesh = plsc.VectorSubcoreMesh(core_axis_name="c", subcore_axis_name="s")
    f = pl.kernel(
        _emb_bag_kernel,
        out_type=jax.ShapeDtypeStruct((B, D), jnp.float32),
        mesh=mesh,
        scratch_types=[
            pltpu.VMEM((2, C, L), jnp.int32),
            pltpu.VMEM((2, C, 72), jnp.int32),
            pltpu.VMEM((2, C * L, 2 * D), jnp.float32),
            pltpu.VMEM((C, D), jnp.float32),
            pltpu.SemaphoreType.DMA((2,)),
        ],
        compiler_params=pltpu.CompilerParams(use_tc_tiling_on_sc=False),
    )
    return f(qs, ho, w2)


def kernel(x, weight):
    return _emb_bag(x.astype(jnp.int32), weight)
